# Initial kernel scaffold; baseline (speedup 1.0000x reference)
import jax
import jax.numpy as jnp
from jax.experimental import pallas as pl


def kernel(f_in, datac, gamma1, beta1, W1, b1, gamma2, beta2, W2, b2):
    raise NotImplementedError("write your pallas kernel here")


# speedup vs baseline: 9.9487x; 9.9487x over previous
import jax
import jax.numpy as jnp
from jax import lax
from jax.experimental import pallas as pl
from jax.experimental.pallas import tpu as pltpu
from jax.experimental.pallas import tpu_sc as plsc

N = 10000
E = 320000
D_IN = 128
D_HID = 256
D_OUT = 128
EPS = 1e-5
NEG_SLOPE = 0.01

NC = 2
NS = 16
NW = NC * NS
CH = 64
KCH = 160
SEG = 2
KSEG = KCH // SEG
EW = CH * KCH
EPAD = EW * NW
NPAD = 10112
RPT = NPAD // NS
DW = 16

_MESH = dict(core_axis_name="c", subcore_axis_name="s", num_cores=NC,
             num_subcores=NS)


def _zero_rows(buf, nrows, width):
    def body(i, _):
        z = jnp.zeros((16,), jnp.float32)
        for k in range(width // 16):
            buf[i, pl.ds(k * 16, 16)] = z
        return 0
    lax.fori_loop(0, nrows, body, 0, unroll=False)


def _zero_acc_slice(zbuf, acc, s):
    base = s * RPT
    for r in range(RPT // CH):
        pltpu.sync_copy(zbuf, acc.at[pl.ds(base + r * CH, CH)])
    rem = RPT % CH
    if rem:
        pltpu.sync_copy(zbuf.at[pl.ds(0, rem)],
                        acc.at[pl.ds(base + (RPT // CH) * CH, rem)])


def _deg_body(didx_hbm, out_hbm, didx_v, ones_v, acc, sem):
    c = lax.axis_index("c")
    s = lax.axis_index("s")
    wid = c * NS + s
    pltpu.async_copy(didx_hbm.at[wid, pl.ds(0, KSEG)], didx_v, sem)
    _zero_rows(ones_v, CH, DW)
    _zero_acc_slice(ones_v, acc, s)
    def fill(i, _):
        ones_v[i, pl.ds(0, 16)] = jnp.ones((16,), jnp.float32)
        return 0
    lax.fori_loop(0, CH, fill, 0, unroll=False)
    pltpu.make_async_copy(didx_hbm.at[wid, pl.ds(0, KSEG)], didx_v,
                          sem).wait()
    plsc.subcore_barrier()

    for seg in range(SEG):
        def body(j, _):
            pltpu.sync_copy(ones_v, acc.at[didx_v.at[j]], add=True)
            return 0
        lax.fori_loop(0, KSEG, body, 0, unroll=False)
        if seg + 1 < SEG:
            pltpu.sync_copy(didx_hbm.at[wid, pl.ds((seg + 1) * KSEG, KSEG)],
                            didx_v)
    plsc.subcore_barrier()
    pltpu.sync_copy(acc.at[pl.ds(s * RPT, RPT)],
                    out_hbm.at[c, pl.ds(s * RPT, RPT)])


def _agg_body(table_hbm, sidx_hbm, didx_hbm, out_hbm,
              sidx_v, didx_v, rows0, rows1, acc, semi, sem0, sem1):
    c = lax.axis_index("c")
    s = lax.axis_index("s")
    wid = c * NS + s
    pltpu.async_copy(sidx_hbm.at[wid, pl.ds(0, KSEG)], sidx_v, semi)
    pltpu.async_copy(didx_hbm.at[wid, pl.ds(0, KSEG)], didx_v, semi)
    _zero_rows(rows0, CH, D_IN)
    _zero_acc_slice(rows0, acc, s)
    pltpu.make_async_copy(sidx_hbm.at[wid, pl.ds(0, KSEG)], sidx_v,
                          semi).wait()
    pltpu.make_async_copy(didx_hbm.at[wid, pl.ds(0, KSEG)], didx_v,
                          semi).wait()
    plsc.subcore_barrier()

    for seg in range(SEG):
        pltpu.async_copy(table_hbm.at[sidx_v.at[0]], rows0, sem0)
        pltpu.async_copy(table_hbm.at[sidx_v.at[1]], rows1, sem1)

        def body(i, _):
            j = 2 * i
            pltpu.make_async_copy(table_hbm.at[sidx_v.at[j]], rows0,
                                  sem0).wait()
            pltpu.sync_copy(rows0, acc.at[didx_v.at[j]], add=True)
            pltpu.async_copy(table_hbm.at[sidx_v.at[j + 2]], rows0, sem0)
            pltpu.make_async_copy(table_hbm.at[sidx_v.at[j + 1]], rows1,
                                  sem1).wait()
            pltpu.sync_copy(rows1, acc.at[didx_v.at[j + 1]], add=True)
            pltpu.async_copy(table_hbm.at[sidx_v.at[j + 3]], rows1, sem1)
            return 0
        lax.fori_loop(0, (KSEG - 2) // 2, body, 0, unroll=False)

        j = KSEG - 2
        pltpu.make_async_copy(table_hbm.at[sidx_v.at[j]], rows0, sem0).wait()
        pltpu.sync_copy(rows0, acc.at[didx_v.at[j]], add=True)
        pltpu.make_async_copy(table_hbm.at[sidx_v.at[j + 1]], rows1,
                              sem1).wait()
        pltpu.sync_copy(rows1, acc.at[didx_v.at[j + 1]], add=True)

        if seg + 1 < SEG:
            pltpu.sync_copy(sidx_hbm.at[wid, pl.ds((seg + 1) * KSEG, KSEG)],
                            sidx_v)
            pltpu.sync_copy(didx_hbm.at[wid, pl.ds((seg + 1) * KSEG, KSEG)],
                            didx_v)

    plsc.subcore_barrier()
    pltpu.sync_copy(acc.at[pl.ds(s * RPT, RPT)],
                    out_hbm.at[c, pl.ds(s * RPT, RPT)])


def _deg_call(dstp):
    return pl.kernel(
        _deg_body,
        out_type=jax.ShapeDtypeStruct((NC, NPAD, DW), jnp.float32),
        mesh=plsc.VectorSubcoreMesh(**_MESH),
        scratch_types=[
            pltpu.VMEM((KSEG, CH), jnp.int32),
            pltpu.VMEM((CH, DW), jnp.float32),
            pltpu.VMEM_SHARED((NPAD, DW), jnp.float32),
            pltpu.SemaphoreType.DMA,
        ],
    )(dstp)


def _agg_call(table, srcp, dstp):
    return pl.kernel(
        _agg_body,
        out_type=jax.ShapeDtypeStruct((NC, NPAD, D_IN), jnp.float32),
        mesh=plsc.VectorSubcoreMesh(**_MESH),
        scratch_types=[
            pltpu.VMEM((KSEG, CH), jnp.int32),
            pltpu.VMEM((KSEG, CH), jnp.int32),
            pltpu.VMEM((CH, D_IN), jnp.float32),
            pltpu.VMEM((CH, D_IN), jnp.float32),
            pltpu.VMEM_SHARED((NPAD, D_IN), jnp.float32),
            pltpu.SemaphoreType.DMA,
            pltpu.SemaphoreType.DMA,
            pltpu.SemaphoreType.DMA,
        ],
    )(table, srcp, dstp)


def _dis_from_parts(degp):
    deg = degp[0, :N, 0:1] + degp[1, :N, 0:1] + 1.0
    return lax.rsqrt(deg)


def _pre_body(f_ref, degp_ref, g1_ref, b1_ref, xs_ref):
    f = f_ref[...]
    mean = jnp.mean(f, axis=0, keepdims=True)
    var = jnp.mean((f - mean) ** 2, axis=0, keepdims=True)
    x = g1_ref[...] * (f - mean) * lax.rsqrt(var + EPS) + b1_ref[...]
    xs_ref[...] = x * _dis_from_parts(degp_ref[...])


def _mid_body(accp_ref, xs_ref, degp_ref, w1_ref, b1_ref, g2_ref, be2_ref,
              w2_ref, ts_ref):
    accp = accp_ref[...]
    dis = _dis_from_parts(degp_ref[...])
    agg = (accp[0, :N, :] + accp[1, :N, :] + xs_ref[...]) * dis
    h = jnp.dot(agg, w1_ref[...], preferred_element_type=jnp.float32)
    h = h + b1_ref[...]
    h = jnp.where(h >= 0, h, NEG_SLOPE * h)
    mean = jnp.mean(h, axis=0, keepdims=True)
    var = jnp.mean((h - mean) ** 2, axis=0, keepdims=True)
    h = g2_ref[...] * (h - mean) * lax.rsqrt(var + EPS) + be2_ref[...]
    t = jnp.dot(h, w2_ref[...], preferred_element_type=jnp.float32)
    ts_ref[...] = t * dis


def _post_body(accp_ref, ts_ref, degp_ref, b2_ref, out_ref):
    accp = accp_ref[...]
    dis = _dis_from_parts(degp_ref[...])
    h2 = (accp[0, :N, :] + accp[1, :N, :] + ts_ref[...]) * dis + b2_ref[...]
    h2 = jnp.where(h2 >= 0, h2, NEG_SLOPE * h2)
    m = jnp.max(h2, axis=1, keepdims=True)
    e = jnp.exp(h2 - m)
    out_ref[...] = e / jnp.sum(e, axis=1, keepdims=True)


def kernel(f_in, datac, gamma1, beta1, W1, b1, gamma2, beta2, W2, b2):
    src = datac[0]
    dst = datac[1]
    pad = EPAD - E
    srcp = jnp.concatenate(
        [src, jnp.zeros((pad,), jnp.int32)]).reshape(NW, KCH, CH)
    dstp = jnp.concatenate(
        [dst, jnp.full((pad,), NPAD - 1, jnp.int32)]).reshape(NW, KCH, CH)

    degp = _deg_call(dstp)

    xs = pl.pallas_call(
        _pre_body,
        out_shape=jax.ShapeDtypeStruct((N, D_IN), jnp.float32),
    )(f_in, degp, gamma1, beta1)

    acc1 = _agg_call(xs, srcp, dstp)

    ts = pl.pallas_call(
        _mid_body,
        out_shape=jax.ShapeDtypeStruct((N, D_OUT), jnp.float32),
    )(acc1, xs, degp, W1, b1, gamma2, beta2, W2)

    acc2 = _agg_call(ts, srcp, dstp)

    out = pl.pallas_call(
        _post_body,
        out_shape=jax.ShapeDtypeStruct((N, D_OUT), jnp.float32),
    )(acc2, ts, degp, b2)
    return out


# speedup vs baseline: 28.5138x; 2.8661x over previous
import jax
import jax.numpy as jnp
from jax import lax
from jax.experimental import pallas as pl
from jax.experimental.pallas import tpu as pltpu
from jax.experimental.pallas import tpu_sc as plsc

N = 10000
E = 320000
D_IN = 128
D_HID = 256
D_OUT = 128
EPS = 1e-5
NEG_SLOPE = 0.01

NC = 2
NS = 16
NW = NC * NS
CH = 64
KCH = 160
SEG = 2
KSEG = KCH // SEG
EW = CH * KCH
EPAD = EW * NW
NPAD = 10112
RPT = NPAD // NS
DW = 16

_MESH = dict(core_axis_name="c", subcore_axis_name="s", num_cores=NC,
             num_subcores=NS)


def _zero_rows(buf, nrows, width):
    def body(i, _):
        z = jnp.zeros((16,), jnp.float32)
        for k in range(width // 16):
            buf[i, pl.ds(k * 16, 16)] = z
        return 0
    lax.fori_loop(0, nrows, body, 0, unroll=False)


def _zero_acc_slice(zbuf, acc, s):
    base = s * RPT
    for r in range(RPT // CH):
        pltpu.sync_copy(zbuf, acc.at[pl.ds(base + r * CH, CH)])
    rem = RPT % CH
    if rem:
        pltpu.sync_copy(zbuf.at[pl.ds(0, rem)],
                        acc.at[pl.ds(base + (RPT // CH) * CH, rem)])


def _deg_body(didx_hbm, out_hbm, didx_v, ones_v, acc, sem):
    c = lax.axis_index("c")
    s = lax.axis_index("s")
    wid = c * NS + s
    pltpu.async_copy(didx_hbm.at[wid, pl.ds(0, KSEG)], didx_v, sem)
    _zero_rows(ones_v, CH, DW)
    _zero_acc_slice(ones_v, acc, s)
    def fill(i, _):
        ones_v[i, pl.ds(0, 16)] = jnp.ones((16,), jnp.float32)
        return 0
    lax.fori_loop(0, CH, fill, 0, unroll=False)
    pltpu.make_async_copy(didx_hbm.at[wid, pl.ds(0, KSEG)], didx_v,
                          sem).wait()
    plsc.subcore_barrier()

    for seg in range(SEG):
        def body(j, _):
            pltpu.sync_copy(ones_v, acc.at[didx_v.at[j]], add=True)
            return 0
        lax.fori_loop(0, KSEG, body, 0, unroll=False)
        if seg + 1 < SEG:
            pltpu.sync_copy(didx_hbm.at[wid, pl.ds((seg + 1) * KSEG, KSEG)],
                            didx_v)
    plsc.subcore_barrier()
    pltpu.sync_copy(acc.at[pl.ds(s * RPT, RPT)],
                    out_hbm.at[c, pl.ds(s * RPT, RPT)])


def _agg_body(table_hbm, sidx_hbm, didx_hbm, out_hbm,
              sidx_v, didx_v, rows0, rows1, acc, semi, sem0, sem1):
    c = lax.axis_index("c")
    s = lax.axis_index("s")
    wid = c * NS + s
    pltpu.async_copy(sidx_hbm.at[wid, pl.ds(0, KSEG)], sidx_v, semi)
    pltpu.async_copy(didx_hbm.at[wid, pl.ds(0, KSEG)], didx_v, semi)
    _zero_rows(rows0, CH, D_IN)
    _zero_acc_slice(rows0, acc, s)
    pltpu.make_async_copy(sidx_hbm.at[wid, pl.ds(0, KSEG)], sidx_v,
                          semi).wait()
    pltpu.make_async_copy(didx_hbm.at[wid, pl.ds(0, KSEG)], didx_v,
                          semi).wait()
    plsc.subcore_barrier()

    for seg in range(SEG):
        pltpu.async_copy(table_hbm.at[sidx_v.at[0]], rows0, sem0)
        pltpu.async_copy(table_hbm.at[sidx_v.at[1]], rows1, sem1)

        def body(i, _):
            j = 2 * i
            pltpu.make_async_copy(table_hbm.at[sidx_v.at[j]], rows0,
                                  sem0).wait()
            pltpu.sync_copy(rows0, acc.at[didx_v.at[j]], add=True)
            pltpu.async_copy(table_hbm.at[sidx_v.at[j + 2]], rows0, sem0)
            pltpu.make_async_copy(table_hbm.at[sidx_v.at[j + 1]], rows1,
                                  sem1).wait()
            pltpu.sync_copy(rows1, acc.at[didx_v.at[j + 1]], add=True)
            pltpu.async_copy(table_hbm.at[sidx_v.at[j + 3]], rows1, sem1)
            return 0
        lax.fori_loop(0, (KSEG - 2) // 2, body, 0, unroll=False)

        j = KSEG - 2
        pltpu.make_async_copy(table_hbm.at[sidx_v.at[j]], rows0, sem0).wait()
        pltpu.sync_copy(rows0, acc.at[didx_v.at[j]], add=True)
        pltpu.make_async_copy(table_hbm.at[sidx_v.at[j + 1]], rows1,
                              sem1).wait()
        pltpu.sync_copy(rows1, acc.at[didx_v.at[j + 1]], add=True)

        if seg + 1 < SEG:
            pltpu.sync_copy(sidx_hbm.at[wid, pl.ds((seg + 1) * KSEG, KSEG)],
                            sidx_v)
            pltpu.sync_copy(didx_hbm.at[wid, pl.ds((seg + 1) * KSEG, KSEG)],
                            didx_v)

    plsc.subcore_barrier()
    pltpu.sync_copy(acc.at[pl.ds(s * RPT, RPT)],
                    out_hbm.at[c, pl.ds(s * RPT, RPT)])


def _deg_call(dstp):
    return pl.kernel(
        _deg_body,
        out_type=jax.ShapeDtypeStruct((NC, NPAD, DW), jnp.float32),
        mesh=plsc.VectorSubcoreMesh(**_MESH),
        scratch_types=[
            pltpu.VMEM((KSEG, CH), jnp.int32),
            pltpu.VMEM((CH, DW), jnp.float32),
            pltpu.VMEM_SHARED((NPAD, DW), jnp.float32),
            pltpu.SemaphoreType.DMA,
        ],
    )(dstp)


def _agg_call(table, srcp, dstp):
    return pl.kernel(
        _agg_body,
        out_type=jax.ShapeDtypeStruct((NC, NPAD, D_IN), jnp.float32),
        mesh=plsc.VectorSubcoreMesh(**_MESH),
        scratch_types=[
            pltpu.VMEM((KSEG, CH), jnp.int32),
            pltpu.VMEM((KSEG, CH), jnp.int32),
            pltpu.VMEM((CH, D_IN), jnp.float32),
            pltpu.VMEM((CH, D_IN), jnp.float32),
            pltpu.VMEM_SHARED((NPAD, D_IN), jnp.float32),
            pltpu.SemaphoreType.DMA,
            pltpu.SemaphoreType.DMA,
            pltpu.SemaphoreType.DMA,
        ],
    )(table, srcp, dstp)


def _dis_from_parts(degp):
    deg = degp[0, :N, 0:1] + degp[1, :N, 0:1] + 1.0
    return lax.rsqrt(deg)


def _pre_body(f_ref, degp_ref, g1_ref, b1_ref, xs_ref):
    f = f_ref[...]
    mean = jnp.mean(f, axis=0, keepdims=True)
    var = jnp.mean((f - mean) ** 2, axis=0, keepdims=True)
    x = g1_ref[...] * (f - mean) * lax.rsqrt(var + EPS) + b1_ref[...]
    xs_ref[...] = x * _dis_from_parts(degp_ref[...])


def _mid_body(accp_ref, xs_ref, degp_ref, w1_ref, b1_ref, g2_ref, be2_ref,
              w2_ref, ts_ref):
    accp = accp_ref[...]
    dis = _dis_from_parts(degp_ref[...])
    agg = (accp[0, :N, :] + accp[1, :N, :] + xs_ref[...]) * dis
    h = jnp.dot(agg, w1_ref[...], preferred_element_type=jnp.float32)
    h = h + b1_ref[...]
    h = jnp.where(h >= 0, h, NEG_SLOPE * h)
    mean = jnp.mean(h, axis=0, keepdims=True)
    var = jnp.mean((h - mean) ** 2, axis=0, keepdims=True)
    h = g2_ref[...] * (h - mean) * lax.rsqrt(var + EPS) + be2_ref[...]
    t = jnp.dot(h, w2_ref[...], preferred_element_type=jnp.float32)
    ts_ref[...] = t * dis


def _post_body(accp_ref, ts_ref, degp_ref, b2_ref, out_ref):
    accp = accp_ref[...]
    dis = _dis_from_parts(degp_ref[...])
    h2 = (accp[0, :N, :] + accp[1, :N, :] + ts_ref[...]) * dis + b2_ref[...]
    h2 = jnp.where(h2 >= 0, h2, NEG_SLOPE * h2)
    m = jnp.max(h2, axis=1, keepdims=True)
    e = jnp.exp(h2 - m)
    out_ref[...] = e / jnp.sum(e, axis=1, keepdims=True)


def kernel(f_in, datac, gamma1, beta1, W1, b1, gamma2, beta2, W2, b2):
    src = datac[0]
    dst = datac[1]
    pad = EPAD - E
    pad_src = jnp.arange(pad, dtype=jnp.int32) % N
    pad_dst = N + jnp.arange(pad, dtype=jnp.int32) % (NPAD - N)
    srcp = jnp.concatenate([src, pad_src]).reshape(NW, KCH, CH)
    dstp = jnp.concatenate([dst, pad_dst]).reshape(NW, KCH, CH)

    degp = _deg_call(dstp)

    xs = pl.pallas_call(
        _pre_body,
        out_shape=jax.ShapeDtypeStruct((N, D_IN), jnp.float32),
    )(f_in, degp, gamma1, beta1)

    acc1 = _agg_call(xs, srcp, dstp)

    ts = pl.pallas_call(
        _mid_body,
        out_shape=jax.ShapeDtypeStruct((N, D_OUT), jnp.float32),
    )(acc1, xs, degp, W1, b1, gamma2, beta2, W2)

    acc2 = _agg_call(ts, srcp, dstp)

    out = pl.pallas_call(
        _post_body,
        out_shape=jax.ShapeDtypeStruct((N, D_OUT), jnp.float32),
    )(acc2, ts, degp, b2)
    return out
